# baseline (speedup 1.0000x reference)
import functools

import jax
import jax.numpy as jnp
from jax import lax
from jax.experimental import pallas as pl
from jax.experimental.pallas import tpu as pltpu
from jax.experimental.pallas import tpu_sc as plsc

_TV = 2000
_NCHUNK = 5


def _sc_info():
  info = plsc.get_sparse_core_info()
  return info.num_cores, info.num_subcores


def _pack_rows(src_ref, dst_ref, nrows):
  ncol = src_ref.shape[1] // 32

  def row_body(r, _):
    def col_body(k, _):
      c = k * 32
      a = src_ref[r, pl.ds(c, 16)]
      b = src_ref[r, pl.ds(c + 16, 16)]
      dst_ref[r, pl.ds(c, 32)] = plsc.pack(
          a, b, format=plsc.PackFormat.INTERLEAVED)
      return 0

    return lax.fori_loop(0, ncol, col_body, 0)

  lax.fori_loop(0, nrows, row_body, 0)


@functools.cache
def _make_sc_gather(V, D, B):
  NC, NS = _sc_info()
  NW = NC * NS
  assert B % (8 * NW) == 0
  b_per_w = B // NW
  mesh = plsc.VectorSubcoreMesh(core_axis_name="c", subcore_axis_name="s")

  @functools.partial(
      pl.kernel,
      mesh=mesh,
      out_type=jax.ShapeDtypeStruct((B, D), jnp.bfloat16),
      scratch_types=[
          pltpu.VMEM((b_per_w,), jnp.int32),
          pltpu.VMEM((b_per_w, D), jnp.float32),
          pltpu.VMEM((b_per_w, D), jnp.bfloat16),
          pltpu.SemaphoreType.DMA,
      ],
      compiler_params=pltpu.CompilerParams(needs_layout_passes=False),
  )
  def sc_gather(table_hbm, idx_hbm, out_hbm, idx_v, rows_v, bf_v, sem):
    wid = lax.axis_index("s") * NC + lax.axis_index("c")
    base = wid * b_per_w
    pltpu.sync_copy(idx_hbm.at[pl.ds(base, b_per_w)], idx_v)
    pltpu.async_copy(table_hbm.at[idx_v], rows_v, sem).wait()
    _pack_rows(rows_v, bf_v, b_per_w)
    pltpu.sync_copy(bf_v, out_hbm.at[pl.ds(base, b_per_w)])

  return sc_gather


_G = 32
_SPAN = 640
_NR = _SPAN // _G


@functools.cache
def _make_sc_wpack(V, D, off, vc):
  NC, NS = _sc_info()
  NW = NC * NS
  assert _SPAN * (NW - 1) < vc <= _SPAN * NW
  mesh = plsc.VectorSubcoreMesh(core_axis_name="c", subcore_axis_name="s")

  @functools.partial(
      pl.kernel,
      mesh=mesh,
      out_type=jax.ShapeDtypeStruct((vc, D), jnp.bfloat16),
      scratch_types=[
          pltpu.VMEM((2, _G, D), jnp.float32),
          pltpu.VMEM((2, _G, D), jnp.bfloat16),
          pltpu.SemaphoreType.DMA,
          pltpu.SemaphoreType.DMA,
          pltpu.SemaphoreType.DMA,
          pltpu.SemaphoreType.DMA,
      ],
      compiler_params=pltpu.CompilerParams(needs_layout_passes=False),
  )
  def sc_wpack(w_hbm, out_hbm, f32_v, bf_v, si0, si1, so0, so1):
    wid = lax.axis_index("s") * NC + lax.axis_index("c")
    base = jnp.minimum(wid * _SPAN, vc - _SPAN)
    sin = (si0, si1)
    sout = (so0, so1)

    def start_in(r, b):
      pltpu.async_copy(
          w_hbm.at[pl.ds(off + base + r * _G, _G)], f32_v.at[b], sin[b])

    def wait_in(b):
      pltpu.make_async_copy(
          w_hbm.at[pl.ds(0, _G)], f32_v.at[b], sin[b]).wait()

    def start_out(r, b):
      pltpu.async_copy(
          bf_v.at[b], out_hbm.at[pl.ds(base + r * _G, _G)], sout[b])

    def wait_out(b):
      pltpu.make_async_copy(
          bf_v.at[b], out_hbm.at[pl.ds(0, _G)], sout[b]).wait()

    start_in(0, 0)
    start_in(1, 1)

    def super_body(s, _):
      for b in range(2):
        r = 2 * s + b
        wait_in(b)

        @pl.when(s >= 1)
        def _():
          wait_out(b)

        def row_body(rr, _):
          def col_body(kk, _):
            c = kk * 32
            a = f32_v[b, rr, pl.ds(c, 16)]
            bb = f32_v[b, rr, pl.ds(c + 16, 16)]
            bf_v[b, rr, pl.ds(c, 32)] = plsc.pack(
                a, bb, format=plsc.PackFormat.INTERLEAVED)
            return 0

          return lax.fori_loop(0, D // 32, col_body, 0)

        lax.fori_loop(0, _G, row_body, 0)
        start_out(r, b)

        @pl.when(s < _NR // 2 - 1)
        def _():
          start_in(r + 2, b)
      return 0

    lax.fori_loop(0, _NR // 2, super_body, 0)
    wait_out(0)
    wait_out(1)

  return sc_wpack


def _mm_body(x_ref, w_ref, o_ref):
  o_ref[...] = lax.dot_general(
      w_ref[...], x_ref[...], (((1,), (1,)), ((), ())),
      preferred_element_type=jnp.float32)


def _mm_body_acc(prev_ref, x_ref, w_ref, o_ref):
  del prev_ref
  _mm_body(x_ref, w_ref, o_ref)


def _mm_chunk(prev, x_bf, w_bf, voff, V):
  T, D = x_bf.shape
  vc = w_bf.shape[0]
  base = voff // _TV
  x_spec = pl.BlockSpec((T, D), lambda i: (0, 0))
  w_spec = pl.BlockSpec((_TV, D), lambda i: (i, 0))
  o_spec = pl.BlockSpec((_TV, T), lambda i: (base + i, 0))
  out_shape = jax.ShapeDtypeStruct((V, T), jnp.float32)
  params = pltpu.CompilerParams(vmem_limit_bytes=100 * 1024 * 1024)
  if prev is None:
    return pl.pallas_call(
        _mm_body,
        grid=(vc // _TV,),
        in_specs=[x_spec, w_spec],
        out_specs=o_spec,
        out_shape=out_shape,
        compiler_params=params,
    )(x_bf, w_bf)
  return pl.pallas_call(
      _mm_body_acc,
      grid=(vc // _TV,),
      in_specs=[pl.BlockSpec(memory_space=pl.ANY), x_spec, w_spec],
      out_specs=o_spec,
      out_shape=out_shape,
      input_output_aliases={0: 0},
      compiler_params=params,
  )(prev, x_bf, w_bf)


def kernel(tokens, w):
  B, T = tokens.shape
  V, D = w.shape
  idx = tokens.reshape(B * T)
  x_bf = _make_sc_gather(V, D, B * T)(w, idx)

  vc = V // _NCHUNK
  w_chunks = [_make_sc_wpack(V, D, c * vc, vc)(w)
              for c in range(_NCHUNK)]

  out = None
  for c in range(_NCHUNK):
    out = _mm_chunk(out, x_bf, w_chunks[c], c * vc, V)
  return out.T.reshape(B, T, V)


# speedup vs baseline: 1.1401x; 1.1401x over previous
import functools

import jax
import jax.numpy as jnp
from jax import lax
from jax.experimental import pallas as pl
from jax.experimental.pallas import tpu as pltpu
from jax.experimental.pallas import tpu_sc as plsc

_TV = 2000
_NCHUNK = 5
_NBF = 3


def _sc_info():
  info = plsc.get_sparse_core_info()
  return info.num_cores, info.num_subcores


def _pack_row(src_ref, dst_ref, b, r, ncol):
  for kk in range(ncol):
    c = kk * 32
    a = src_ref[b, r, pl.ds(c, 16)]
    bb = src_ref[b, r, pl.ds(c + 16, 16)]
    dst_ref[b, r, pl.ds(c, 32)] = plsc.pack(
        a, bb, format=plsc.PackFormat.INTERLEAVED)


@functools.cache
def _make_sc_gather(V, D, B):
  NC, NS = _sc_info()
  NW = NC * NS
  assert B % (8 * NW) == 0
  b_per_w = B // NW
  mesh = plsc.VectorSubcoreMesh(core_axis_name="c", subcore_axis_name="s")

  @functools.partial(
      pl.kernel,
      mesh=mesh,
      out_type=(jax.ShapeDtypeStruct((B, D), jnp.float32),
                jax.ShapeDtypeStruct((B, D), jnp.bfloat16)),
      scratch_types=[
          pltpu.VMEM((b_per_w,), jnp.int32),
          pltpu.VMEM((1, b_per_w, D), jnp.float32),
          pltpu.VMEM((1, b_per_w, D), jnp.bfloat16),
          pltpu.SemaphoreType.DMA,
      ],
      compiler_params=pltpu.CompilerParams(needs_layout_passes=False),
  )
  def sc_gather(table_hbm, idx_hbm, out32_hbm, outbf_hbm, idx_v, rows_v,
                bf_v, sem):
    wid = lax.axis_index("s") * NC + lax.axis_index("c")
    base = wid * b_per_w
    pltpu.sync_copy(idx_hbm.at[pl.ds(base, b_per_w)], idx_v)
    pltpu.async_copy(table_hbm.at[idx_v], rows_v.at[0], sem).wait()
    pltpu.sync_copy(rows_v.at[0], out32_hbm.at[pl.ds(base, b_per_w)])

    def row_body(r, _):
      _pack_row(rows_v, bf_v, 0, r, D // 32)
      return 0

    lax.fori_loop(0, b_per_w, row_body, 0)
    pltpu.sync_copy(bf_v.at[0], outbf_hbm.at[pl.ds(base, b_per_w)])

  return sc_gather


_G = 32
_SPAN = 640
_NR = _SPAN // _G


@functools.cache
def _make_sc_wpack(V, D, off, vc):
  NC, NS = _sc_info()
  NW = NC * NS
  assert _SPAN * (NW - 1) < vc <= _SPAN * NW
  mesh = plsc.VectorSubcoreMesh(core_axis_name="c", subcore_axis_name="s")

  @functools.partial(
      pl.kernel,
      mesh=mesh,
      out_type=jax.ShapeDtypeStruct((vc, D), jnp.bfloat16),
      scratch_types=[
          pltpu.VMEM((2, _G, D), jnp.float32),
          pltpu.VMEM((2, _G, D), jnp.bfloat16),
          pltpu.SemaphoreType.DMA,
          pltpu.SemaphoreType.DMA,
          pltpu.SemaphoreType.DMA,
          pltpu.SemaphoreType.DMA,
      ],
      compiler_params=pltpu.CompilerParams(needs_layout_passes=False),
  )
  def sc_wpack(w_hbm, out_hbm, f32_v, bf_v, si0, si1, so0, so1):
    wid = lax.axis_index("s") * NC + lax.axis_index("c")
    base = jnp.minimum(wid * _SPAN, vc - _SPAN)
    sin = (si0, si1)
    sout = (so0, so1)

    def start_in(r, b):
      pltpu.async_copy(
          w_hbm.at[pl.ds(off + base + r * _G, _G)], f32_v.at[b], sin[b])

    def wait_in(b):
      pltpu.make_async_copy(
          w_hbm.at[pl.ds(0, _G)], f32_v.at[b], sin[b]).wait()

    def start_out(r, b):
      pltpu.async_copy(
          bf_v.at[b], out_hbm.at[pl.ds(base + r * _G, _G)], sout[b])

    def wait_out(b):
      pltpu.make_async_copy(
          bf_v.at[b], out_hbm.at[pl.ds(0, _G)], sout[b]).wait()

    start_in(0, 0)
    start_in(1, 1)

    def super_body(s, _):
      for b in range(2):
        r = 2 * s + b
        wait_in(b)

        @pl.when(s >= 1)
        def _():
          wait_out(b)

        def row_body(rr, _):
          _pack_row(f32_v, bf_v, b, rr, D // 32)
          return 0

        lax.fori_loop(0, _G, row_body, 0)
        start_out(r, b)

        @pl.when(s < _NR // 2 - 1)
        def _():
          start_in(r + 2, b)
      return 0

    lax.fori_loop(0, _NR // 2, super_body, 0)
    wait_out(0)
    wait_out(1)

  return sc_wpack


def _mm_body_bf(x_ref, w_ref, o_ref):
  o_ref[...] = lax.dot_general(
      w_ref[...], x_ref[...], (((1,), (1,)), ((), ())),
      preferred_element_type=jnp.float32)


def _mm_body_f32(x_ref, w_ref, o_ref):
  wb = w_ref[...].astype(jnp.bfloat16)
  o_ref[...] = lax.dot_general(
      wb, x_ref[...], (((1,), (1,)), ((), ())),
      preferred_element_type=jnp.float32)


def _mm_body_bf_acc(prev_ref, x_ref, w_ref, o_ref):
  del prev_ref
  _mm_body_bf(x_ref, w_ref, o_ref)


def _mm_body_f32_acc(prev_ref, x_ref, w_ref, o_ref):
  del prev_ref
  _mm_body_f32(x_ref, w_ref, o_ref)


def _mm_chunk(prev, x_bf, w_arr, woff, voff, vc, V):
  T, D = x_bf.shape
  base_o = voff // _TV
  base_w = woff // _TV
  is_bf = w_arr.dtype == jnp.bfloat16
  body = _mm_body_bf if is_bf else _mm_body_f32
  body_acc = _mm_body_bf_acc if is_bf else _mm_body_f32_acc
  x_spec = pl.BlockSpec((T, D), lambda i: (0, 0))
  w_spec = pl.BlockSpec((_TV, D), lambda i: (base_w + i, 0))
  o_spec = pl.BlockSpec((_TV, T), lambda i: (base_o + i, 0))
  out_shape = jax.ShapeDtypeStruct((V, T), jnp.float32)
  params = pltpu.CompilerParams(vmem_limit_bytes=100 * 1024 * 1024)
  if prev is None:
    return pl.pallas_call(
        body,
        grid=(vc // _TV,),
        in_specs=[x_spec, w_spec],
        out_specs=o_spec,
        out_shape=out_shape,
        compiler_params=params,
    )(x_bf, w_arr)
  return pl.pallas_call(
      body_acc,
      grid=(vc // _TV,),
      in_specs=[pl.BlockSpec(memory_space=pl.ANY), x_spec, w_spec],
      out_specs=o_spec,
      out_shape=out_shape,
      input_output_aliases={0: 0},
      compiler_params=params,
  )(prev, x_bf, w_arr)


def kernel(tokens, w):
  B, T = tokens.shape
  V, D = w.shape
  idx = tokens.reshape(B * T)
  x32, x_pi = _make_sc_gather(V, D, B * T)(w, idx)
  x_nat = x32.astype(jnp.bfloat16)

  vc = V // _NCHUNK
  w_bf = [_make_sc_wpack(V, D, c * vc, vc)(w) for c in range(_NBF)]

  out = None
  for c in range(_NBF, _NCHUNK):
    out = _mm_chunk(out, x_nat, w, c * vc, c * vc, vc, V)
  for c in range(_NBF):
    out = _mm_chunk(out, x_pi, w_bf[c], 0, c * vc, vc, V)
  return out.T.reshape(B, T, V)


# speedup vs baseline: 1.4728x; 1.2918x over previous
import functools

import jax
import jax.numpy as jnp
from jax import lax
from jax.experimental import pallas as pl
from jax.experimental.pallas import tpu as pltpu
from jax.experimental.pallas import tpu_sc as plsc


@functools.cache
def _make_sc_gather(V, D, B):
  info = plsc.get_sparse_core_info()
  NC, NS = info.num_cores, info.num_subcores
  NW = NC * NS
  assert B % (8 * NW) == 0 and D % info.num_lanes == 0
  b_per_w = B // NW
  mesh = plsc.VectorSubcoreMesh(core_axis_name="c", subcore_axis_name="s")

  @functools.partial(
      pl.kernel,
      mesh=mesh,
      out_type=jax.ShapeDtypeStruct((B, D), jnp.float32),
      scratch_types=[
          pltpu.VMEM((b_per_w,), jnp.int32),
          pltpu.VMEM((b_per_w, D), jnp.float32),
          pltpu.SemaphoreType.DMA,
      ],
  )
  def sc_gather(table_hbm, idx_hbm, out_hbm, idx_v, rows_v, sem):
    wid = lax.axis_index("s") * NC + lax.axis_index("c")
    base = wid * b_per_w
    pltpu.sync_copy(idx_hbm.at[pl.ds(base, b_per_w)], idx_v)
    pltpu.async_copy(table_hbm.at[idx_v], rows_v, sem).wait()
    pltpu.sync_copy(rows_v, out_hbm.at[pl.ds(base, b_per_w)])

  return sc_gather


_TV = 2000


def _mm_body(x_ref, w_ref, o_ref):
  o_ref[...] = lax.dot_general(
      w_ref[...], x_ref[...], (((1,), (1,)), ((), ())),
      preferred_element_type=jnp.float32)


def _matmul_t(x_bf, w):
  T, D = x_bf.shape
  V = w.shape[0]
  return pl.pallas_call(
      _mm_body,
      grid=(V // _TV,),
      in_specs=[
          pl.BlockSpec((T, D), lambda i: (0, 0)),
          pl.BlockSpec((_TV, D), lambda i: (i, 0)),
      ],
      out_specs=pl.BlockSpec((_TV, T), lambda i: (i, 0)),
      out_shape=jax.ShapeDtypeStruct((V, T), jnp.float32),
      compiler_params=pltpu.CompilerParams(
          vmem_limit_bytes=100 * 1024 * 1024),
  )(x_bf, w)


def kernel(tokens, w):
  B, T = tokens.shape
  V, D = w.shape
  idx = tokens.reshape(B * T)
  x = _make_sc_gather(V, D, B * T)(w, idx)
  x_bf = x.astype(jnp.bfloat16)
  logits_t = _matmul_t(x_bf, w)
  return logits_t.T.reshape(B, T, V)
